# Initial kernel scaffold; baseline (speedup 1.0000x reference)
import jax
import jax.numpy as jnp
from jax.experimental import pallas as pl


def kernel(x, codebook_weight):
    raise NotImplementedError("write your pallas kernel here")


# speedup vs baseline: 2.1832x; 2.1832x over previous
import jax
import jax.numpy as jnp
from jax.experimental import pallas as pl
from jax.experimental.pallas import tpu as pltpu

K = 8192
D = 64
B = 8192
BLK = 256
MU = 0.25


def _vq_block_kernel(x_ref, cb_ref, quant_ref, idx_ref, probs_ref, loss_ref):
    x = x_ref[...]
    cb = cb_ref[...]
    mm = jnp.dot(x, cb.T, preferred_element_type=jnp.float32)
    xn = jnp.sum(x * x, axis=-1, keepdims=True)
    cn = jnp.sum(cb * cb, axis=-1, keepdims=True).T
    d2 = xn + cn - 2.0 * mm
    nd = -d2
    m = jnp.max(nd, axis=-1, keepdims=True)
    e = jnp.exp(nd - m)
    probs = e / jnp.sum(e, axis=-1, keepdims=True)
    probs_ref[...] = probs
    idx = jnp.argmax(probs, axis=-1)
    idx_ref[...] = idx.astype(jnp.int32)
    loss_ref[...] = ((1.0 + MU) / D) * (-m[:, 0])
    iota = jax.lax.broadcasted_iota(jnp.int32, (BLK, K), 1)
    onehot = (iota == idx[:, None]).astype(jnp.float32)
    quant_ref[...] = jnp.dot(onehot, cb, preferred_element_type=jnp.float32)


def kernel(x, codebook_weight):
    grid = (B // BLK,)
    quant, idx, probs, loss = pl.pallas_call(
        _vq_block_kernel,
        grid=grid,
        in_specs=[
            pl.BlockSpec((BLK, D), lambda i: (i, 0)),
            pl.BlockSpec((K, D), lambda i: (0, 0)),
        ],
        out_specs=[
            pl.BlockSpec((BLK, D), lambda i: (i, 0)),
            pl.BlockSpec((BLK,), lambda i: (i,)),
            pl.BlockSpec((BLK, K), lambda i: (i, 0)),
            pl.BlockSpec((BLK,), lambda i: (i,)),
        ],
        out_shape=[
            jax.ShapeDtypeStruct((B, D), jnp.float32),
            jax.ShapeDtypeStruct((B,), jnp.int32),
            jax.ShapeDtypeStruct((B, K), jnp.float32),
            jax.ShapeDtypeStruct((B,), jnp.float32),
        ],
        compiler_params=pltpu.CompilerParams(
            dimension_semantics=("arbitrary",),
        ),
    )(x, codebook_weight)
    return (quant, idx, probs, loss)


# speedup vs baseline: 3.4640x; 1.5867x over previous
import functools

import jax
import jax.numpy as jnp
from jax import lax
from jax.experimental import pallas as pl
from jax.experimental.pallas import tpu as pltpu
from jax.experimental.pallas import tpu_sc as plsc

K = 8192
D = 64
B = 8192
BLK = 256
MU = 0.25


def _vq_block_kernel(x_ref, cb_ref, idx_ref, probs_ref, loss_ref, cn_ref):
    i = pl.program_id(0)

    @pl.when(i == 0)
    def _():
        cb = cb_ref[...]
        ones = jnp.ones((1, D), jnp.float32)
        cn_ref[...] = lax.dot_general(
            ones, cb * cb, (((1,), (1,)), ((), ())),
            precision=lax.Precision.HIGHEST,
            preferred_element_type=jnp.float32)

    x = x_ref[...]
    mm = lax.dot_general(x, cb_ref[...], (((1,), (1,)), ((), ())),
                         preferred_element_type=jnp.float32)
    u = 2.0 * mm - cn_ref[...]
    mu = jnp.max(u, axis=-1, keepdims=True)
    e = jnp.exp(u - mu)
    r = 1.0 / jnp.sum(e, axis=-1, keepdims=True)
    probs = e * r
    probs_ref[...] = probs
    iota = lax.broadcasted_iota(jnp.int32, (BLK, K), 1)
    cand = jnp.where(probs == r, iota, K)
    idx_ref[...] = jnp.min(cand, axis=-1)
    xn = jnp.sum(x * x, axis=-1)
    loss_ref[...] = ((1.0 + MU) / D) * (xn - mu[:, 0])


def _distances_softmax(x, codebook_weight):
    grid = (B // BLK,)
    idx, probs, loss = pl.pallas_call(
        _vq_block_kernel,
        grid=grid,
        in_specs=[
            pl.BlockSpec((BLK, D), lambda i: (i, 0)),
            pl.BlockSpec((K, D), lambda i: (0, 0)),
        ],
        out_specs=[
            pl.BlockSpec((BLK,), lambda i: (i,)),
            pl.BlockSpec((BLK, K), lambda i: (i, 0)),
            pl.BlockSpec((BLK,), lambda i: (i,)),
        ],
        out_shape=[
            jax.ShapeDtypeStruct((B,), jnp.int32),
            jax.ShapeDtypeStruct((B, K), jnp.float32),
            jax.ShapeDtypeStruct((B,), jnp.float32),
        ],
        scratch_shapes=[pltpu.VMEM((1, K), jnp.float32)],
        compiler_params=pltpu.CompilerParams(
            dimension_semantics=("arbitrary",),
        ),
    )(x, codebook_weight)
    return idx, probs, loss


_SC_INFO = plsc.get_sparse_core_info()
_NW = _SC_INFO.num_cores * _SC_INFO.num_subcores
_B_PER_W = B // _NW
_DP = 128


@functools.partial(
    pl.kernel,
    mesh=plsc.VectorSubcoreMesh(core_axis_name="c", subcore_axis_name="s"),
    out_type=jax.ShapeDtypeStruct((B, _DP), jnp.float32),
    scratch_types=[
        pltpu.VMEM((_B_PER_W,), jnp.int32),
        pltpu.VMEM((_B_PER_W, _DP), jnp.float32),
        pltpu.SemaphoreType.DMA,
    ],
)
def _sc_gather(table_hbm, idx_hbm, out_hbm, idx_v, rows_v, sem):
    wid = lax.axis_index("s") * _SC_INFO.num_cores + lax.axis_index("c")
    base = wid * _B_PER_W
    pltpu.sync_copy(idx_hbm.at[pl.ds(base, _B_PER_W)], idx_v)
    pltpu.async_copy(table_hbm.at[idx_v], rows_v, sem).wait()
    pltpu.sync_copy(rows_v, out_hbm.at[pl.ds(base, _B_PER_W)])


def kernel(x, codebook_weight):
    idx, probs, loss = _distances_softmax(x, codebook_weight)
    table = jnp.pad(codebook_weight, ((0, 0), (0, _DP - D)))
    quant = _sc_gather(table, idx)[:, :D]
    return (quant, idx, probs, loss)
